# baseline (speedup 1.0000x reference)
import functools

import numpy as np
import jax
import jax.numpy as jnp
from jax import lax
from jax.experimental import pallas as pl
from jax.experimental.pallas import tpu as pltpu
from jax.experimental.pallas import tpu_sc as plsc

_VOCAB = 100000
_EMBED = 128
_WINDOW = 200
_BATCH = 1024
_SCALE = float(np.sqrt(float(_EMBED)))

_NC = 2
_NS = 16
_NW = _NC * _NS
_ROWS_PER_W = _BATCH // _NW
_HALF = _WINDOW // 2
_PAIRS = _ROWS_PER_W // 2
_C0 = 104
_C1 = _WINDOW - _C0


def _positional_encoding(length, depth):
    pos = np.arange(length)[:, np.newaxis]
    i = np.arange(depth)[np.newaxis, :]
    val = pos / 10000 ** (2 * (i // 2) / depth)
    pe = np.concatenate([np.sin(val[:, 0::2]), np.cos(val[:, 1::2])], axis=-1)
    return pe.astype(np.float32)


_POS = _positional_encoding(_WINDOW, _EMBED)


def _sc_body(x_hbm, pos_hbm, table_hbm, out_hbm,
             idx_v, in0, in1, out0, out1, pos_v, sg0, sg1, sw0, sw1):
    ins = (in0, in1)
    outs = (out0, out1)
    offs = (0, _C0)
    sizes = (_C0, _C1)
    sgs = (sg0, sg1)
    sws = (sw0, sw1)
    wid = lax.axis_index("s") * _NC + lax.axis_index("c")
    base = wid * _ROWS_PER_W
    pltpu.sync_copy(pos_hbm, pos_v)
    pltpu.sync_copy(x_hbm.at[wid], idx_v)

    def start_gather_half(r, s, h):
        pltpu.async_copy(table_hbm.at[idx_v.at[r, h]],
                         ins[s].at[pl.ds(h * _HALF, _HALF)], sgs[s])

    def start_gather(r, s):
        start_gather_half(r, s, 0)
        start_gather_half(r, s, 1)

    def wait_gather(s):
        pltpu.make_async_copy(
            table_hbm.at[pl.ds(0, _WINDOW)], ins[s], sgs[s]).wait()

    def start_wb(q, r):
        pltpu.async_copy(outs[q],
                         out_hbm.at[base + r, pl.ds(offs[q], sizes[q])],
                         sws[q])

    def wait_wb(q):
        pltpu.make_async_copy(
            outs[q], out_hbm.at[0, pl.ds(offs[q], sizes[q])], sws[q]).wait()

    def compute_chunk(s, q):
        src = ins[s]
        dst = outs[q]
        off = offs[q]

        def tok(t, c):
            for u in range(4):
                tt = t * 4 + u
                for v in range(_EMBED // 16):
                    dst[tt, pl.ds(v * 16, 16)] = (
                        src[off + tt, pl.ds(v * 16, 16)] * _SCALE
                        + pos_v[off + tt, pl.ds(v * 16, 16)])
            return c
        lax.fori_loop(0, sizes[q] // 4, tok, 0)

    start_gather(0, 0)
    start_gather(1, 1)

    def pair(j, carry):
        for s in range(2):
            r = 2 * j + s
            wait_gather(s)
            for q in range(2):

                @pl.when(r > 0)
                def _():
                    wait_wb(q)
                compute_chunk(s, q)
                start_wb(q, r)

                @pl.when(j < _PAIRS - 1)
                def _():
                    start_gather_half(r + 2, s, q)
        return carry

    lax.fori_loop(0, _PAIRS, pair, 0)
    wait_wb(0)
    wait_wb(1)


@jax.jit
def kernel(x, table):
    x4 = x.reshape(_NW, _ROWS_PER_W, 2, _HALF)
    pos = jnp.asarray(_POS)
    mesh = plsc.VectorSubcoreMesh(core_axis_name="c", subcore_axis_name="s")
    call = functools.partial(
        pl.kernel,
        mesh=mesh,
        out_type=jax.ShapeDtypeStruct((_BATCH, _WINDOW, _EMBED), jnp.float32),
        scratch_types=(
            [pltpu.VMEM((_ROWS_PER_W, 2, _HALF), jnp.int32)]
            + [pltpu.VMEM((_WINDOW, _EMBED), jnp.float32)] * 2
            + [pltpu.VMEM((_C0, _EMBED), jnp.float32)]
            + [pltpu.VMEM((_C1, _EMBED), jnp.float32)]
            + [pltpu.VMEM((_WINDOW, _EMBED), jnp.float32)]
            + [pltpu.SemaphoreType.DMA] * 4
        ),
    )(_sc_body)
    return call(x4, pos, table)


# speedup vs baseline: 1.0021x; 1.0021x over previous
import functools

import numpy as np
import jax
import jax.numpy as jnp
from jax import lax
from jax.experimental import pallas as pl
from jax.experimental.pallas import tpu as pltpu
from jax.experimental.pallas import tpu_sc as plsc

_VOCAB = 100000
_EMBED = 128
_WINDOW = 200
_BATCH = 1024
_SCALE = float(np.sqrt(float(_EMBED)))

_NC = 2
_NS = 16
_NW = _NC * _NS
_ROWS_PER_W = _BATCH // _NW
_HALF = _WINDOW // 2
_PAIRS = _ROWS_PER_W // 2
_C0 = 104
_C1 = _WINDOW - _C0


def _positional_encoding(length, depth):
    pos = np.arange(length)[:, np.newaxis]
    i = np.arange(depth)[np.newaxis, :]
    val = pos / 10000 ** (2 * (i // 2) / depth)
    pe = np.concatenate([np.sin(val[:, 0::2]), np.cos(val[:, 1::2])], axis=-1)
    return pe.astype(np.float32)


_POS = _positional_encoding(_WINDOW, _EMBED)


def _sc_body(x_hbm, pos_hbm, table_hbm, out_hbm,
             idx_v, in0, in1, out0, out1, pos_v, sg0, sg1, sw0, sw1):
    ins = (in0, in1)
    outs = (out0, out1)
    offs = (0, _C0)
    sizes = (_C0, _C1)
    sgs = (sg0, sg1)
    sws = (sw0, sw1)
    wid = lax.axis_index("s") * _NC + lax.axis_index("c")
    base = wid * _ROWS_PER_W
    pltpu.sync_copy(pos_hbm, pos_v)
    pltpu.sync_copy(x_hbm.at[wid], idx_v)

    def start_gather_half(r, s, h):
        pltpu.async_copy(table_hbm.at[idx_v.at[r, h]],
                         ins[s].at[pl.ds(h * _HALF, _HALF)], sgs[s])

    def start_gather(r, s):
        start_gather_half(r, s, 0)
        start_gather_half(r, s, 1)

    def wait_gather(s):
        pltpu.make_async_copy(
            table_hbm.at[pl.ds(0, _WINDOW)], ins[s], sgs[s]).wait()

    def start_wb(q, r):
        pltpu.async_copy(outs[q],
                         out_hbm.at[base + r, pl.ds(offs[q], sizes[q])],
                         sws[q])

    def wait_wb(q):
        pltpu.make_async_copy(
            outs[q], out_hbm.at[0, pl.ds(offs[q], sizes[q])], sws[q]).wait()

    def compute_chunk(s, q):
        src = ins[s]
        dst = outs[q]
        off = offs[q]

        def tok(t, c):
            for u in range(2):
                tt = t * 2 + u
                for v in range(_EMBED // 16):
                    dst[tt, pl.ds(v * 16, 16)] = (
                        src[off + tt, pl.ds(v * 16, 16)] * _SCALE
                        + pos_v[off + tt, pl.ds(v * 16, 16)])
            return c
        lax.fori_loop(0, sizes[q] // 2, tok, 0)

    start_gather(0, 0)
    start_gather(1, 1)

    def pair(j, carry):
        for s in range(2):
            r = 2 * j + s
            wait_gather(s)
            for q in range(2):

                @pl.when(r > 0)
                def _():
                    wait_wb(q)
                compute_chunk(s, q)
                start_wb(q, r)

                @pl.when(j < _PAIRS - 1)
                def _():
                    start_gather_half(r + 2, s, q)
        return carry

    lax.fori_loop(0, _PAIRS, pair, 0)
    wait_wb(0)
    wait_wb(1)


@jax.jit
def kernel(x, table):
    x4 = x.reshape(_NW, _ROWS_PER_W, 2, _HALF)
    pos = jnp.asarray(_POS)
    mesh = plsc.VectorSubcoreMesh(core_axis_name="c", subcore_axis_name="s")
    call = functools.partial(
        pl.kernel,
        mesh=mesh,
        out_type=jax.ShapeDtypeStruct((_BATCH, _WINDOW, _EMBED), jnp.float32),
        scratch_types=(
            [pltpu.VMEM((_ROWS_PER_W, 2, _HALF), jnp.int32)]
            + [pltpu.VMEM((_WINDOW, _EMBED), jnp.float32)] * 2
            + [pltpu.VMEM((_C0, _EMBED), jnp.float32)]
            + [pltpu.VMEM((_C1, _EMBED), jnp.float32)]
            + [pltpu.VMEM((_WINDOW, _EMBED), jnp.float32)]
            + [pltpu.SemaphoreType.DMA] * 4
        ),
    )(_sc_body)
    return call(x4, pos, table)
